# baseline (speedup 1.0000x reference)
import functools

import jax
import jax.numpy as jnp
from jax import lax
from jax.experimental import pallas as pl
from jax.experimental.pallas import tpu as pltpu
from jax.experimental.pallas import tpu_sc as plsc

N_NODES = 10000
N_PAD = 10240
D = 128
C = 128
E = 320000
NC, NS = 2, 16
NW = NC * NS
CHUNK = 128
NCH = 80
E_PAD = (NCH + 1) * NW * CHUNK
ROWS_PER_TILE = N_PAD // NS
BLK = 1024


def _prep_body(x_ref, wt_ref, vs_ref, vd_ref, xs_ref, asrc_ref, adst_ref):
    xs = jnp.dot(x_ref[...], wt_ref[...], preferred_element_type=jnp.float32)
    xs_ref[...] = xs
    asrc_ref[...] = jnp.sum(xs * vs_ref[...][None, :], axis=1)
    adst_ref[...] = jnp.sum(xs * vd_ref[...][None, :], axis=1)


def _prep(x_pad, wt, vs, vd):
    grid = (N_PAD // BLK,)
    return pl.pallas_call(
        _prep_body,
        grid=grid,
        in_specs=[
            pl.BlockSpec((BLK, D), lambda i: (i, 0)),
            pl.BlockSpec((D, C), lambda i: (0, 0)),
            pl.BlockSpec((C,), lambda i: (0,)),
            pl.BlockSpec((C,), lambda i: (0,)),
        ],
        out_specs=[
            pl.BlockSpec((BLK, C), lambda i: (i, 0)),
            pl.BlockSpec((BLK,), lambda i: (i,)),
            pl.BlockSpec((BLK,), lambda i: (i,)),
        ],
        out_shape=[
            jax.ShapeDtypeStruct((N_PAD, C), jnp.float32),
            jax.ShapeDtypeStruct((N_PAD,), jnp.float32),
            jax.ShapeDtypeStruct((N_PAD,), jnp.float32),
        ],
    )(x_pad, wt, vs, vd)


def _sc_body(xs_hbm, asrc_hbm, adst_hbm, edges_hbm,
             acc_out, den_out,
             idx0_v, idx1_v, as0_v, as1_v, ad0_v, ad1_v, e0_v, e1_v,
             rows0_v, rows1_v, acc_sh, den_sh, sem_g0, sem_g1):
    cid = lax.axis_index("c")
    sid = lax.axis_index("s")
    wid = cid * NS + sid
    idx2 = (idx0_v, idx1_v)
    asv = (as0_v, as1_v)
    adv = (ad0_v, ad1_v)
    ev = (e0_v, e1_v)
    rows = (rows0_v, rows1_v)
    sem_g = (sem_g0, sem_g1)

    zeros16 = jnp.zeros((16,), jnp.float32)

    def _zrow(i, carry):
        for q in range(C // 16):
            rows0_v[i, pl.ds(q * 16, 16)] = zeros16
        return carry

    lax.fori_loop(0, CHUNK, _zrow, 0)
    for q in range(CHUNK // 16):
        e0_v[pl.ds(q * 16, 16)] = zeros16
    for t in range(ROWS_PER_TILE // CHUNK):
        r0 = sid * ROWS_PER_TILE + t * CHUNK
        pltpu.sync_copy(rows0_v, acc_sh.at[pl.ds(r0, CHUNK)])
        pltpu.sync_copy(e0_v, den_sh.at[pl.ds(r0, CHUNK)])
    plsc.subcore_barrier()

    def _issue_gathers(b, g):
        base = (wid + NW * g) * CHUNK
        pltpu.sync_copy(edges_hbm.at[:, pl.ds(base, CHUNK)], idx2[b])
        pltpu.async_copy(xs_hbm.at[idx2[b].at[0]], rows[b], sem_g[b])
        pltpu.async_copy(asrc_hbm.at[idx2[b].at[0]], asv[b], sem_g[b])
        pltpu.async_copy(adst_hbm.at[idx2[b].at[1]], adv[b], sem_g[b])

    def _wait_gathers(b):
        pltpu.make_async_copy(xs_hbm.at[idx2[b].at[0]], rows[b], sem_g[b]).wait()
        pltpu.make_async_copy(asrc_hbm.at[idx2[b].at[0]], asv[b], sem_g[b]).wait()
        pltpu.make_async_copy(adst_hbm.at[idx2[b].at[1]], adv[b], sem_g[b]).wait()

    def _process(b):
        for g in range(CHUNK // 16):
            sl = pl.ds(g * 16, 16)
            si = idx2[b][0, sl]
            di = idx2[b][1, sl]
            s = asv[b][sl] + adv[b][sl]
            s = jnp.where(s >= 0.0, s, 0.2 * s)
            ev[b][sl] = jnp.where(si != di, jnp.exp(s), 0.0)

        def _scale(it, carry):
            for u in range(4):
                i = it * 4 + u
                spl = plsc.load_gather(
                    ev[b], [jnp.full((16,), 0, jnp.int32) + i])
                for q in range(C // 16):
                    sl = pl.ds(q * 16, 16)
                    rows[b][i, sl] = rows[b][i, sl] * spl
            return carry

        lax.fori_loop(0, CHUNK // 4, _scale, 0)
        pltpu.sync_copy(rows[b], acc_sh.at[idx2[b].at[1]], add=True)
        pltpu.sync_copy(ev[b], den_sh.at[idx2[b].at[1]], add=True)

    _issue_gathers(0, 0)

    def _pair(jj, carry):
        for b in range(2):
            g = jj * 2 + b
            _issue_gathers(1 - b, g + 1)
            _wait_gathers(b)
            _process(b)
        return carry

    lax.fori_loop(0, NCH // 2, _pair, 0)
    _wait_gathers(0)
    plsc.subcore_barrier()

    for t in range(ROWS_PER_TILE // CHUNK):
        r0 = sid * ROWS_PER_TILE + t * CHUNK
        pltpu.sync_copy(acc_sh.at[pl.ds(r0, CHUNK)], rows0_v)
        pltpu.sync_copy(rows0_v, acc_out.at[cid, pl.ds(r0, CHUNK)])
        pltpu.sync_copy(den_sh.at[pl.ds(r0, CHUNK)], e0_v)
        pltpu.sync_copy(e0_v, den_out.at[cid, pl.ds(r0, CHUNK)])


_sc_edges = functools.partial(
    pl.kernel,
    out_type=[
        jax.ShapeDtypeStruct((NC, N_PAD, C), jnp.float32),
        jax.ShapeDtypeStruct((NC, N_PAD), jnp.float32),
    ],
    mesh=plsc.VectorSubcoreMesh(core_axis_name="c", subcore_axis_name="s"),
    compiler_params=pltpu.CompilerParams(
        needs_layout_passes=False, use_tc_tiling_on_sc=False),
    scratch_types=[
        pltpu.VMEM((2, CHUNK), jnp.int32),
        pltpu.VMEM((2, CHUNK), jnp.int32),
        pltpu.VMEM((CHUNK,), jnp.float32),
        pltpu.VMEM((CHUNK,), jnp.float32),
        pltpu.VMEM((CHUNK,), jnp.float32),
        pltpu.VMEM((CHUNK,), jnp.float32),
        pltpu.VMEM((CHUNK,), jnp.float32),
        pltpu.VMEM((CHUNK,), jnp.float32),
        pltpu.VMEM((CHUNK, C), jnp.float32),
        pltpu.VMEM((CHUNK, C), jnp.float32),
        pltpu.VMEM_SHARED((N_PAD, C), jnp.float32),
        pltpu.VMEM_SHARED((N_PAD,), jnp.float32),
        pltpu.SemaphoreType.DMA,
        pltpu.SemaphoreType.DMA,
    ],
)(_sc_body)


def _combine_body(acc0_ref, acc1_ref, den0_ref, den1_ref, asrc_ref, adst_ref,
                  xs_ref, bias_ref, out_ref):
    s = asrc_ref[...] + adst_ref[...]
    s = jnp.where(s >= 0.0, s, 0.2 * s)
    es = jnp.exp(s)
    den = den0_ref[...] + den1_ref[...] + es
    num = acc0_ref[...] + acc1_ref[...] + es[:, None] * xs_ref[...]
    out_ref[...] = num / (den[:, None] + 1e-16) + bias_ref[...][None, :]


def _combine(acc0, acc1, den0, den1, asrc, adst, xs, bias):
    grid = (N_PAD // BLK,)
    mat = pl.BlockSpec((BLK, C), lambda i: (i, 0))
    vec = pl.BlockSpec((BLK,), lambda i: (i,))
    return pl.pallas_call(
        _combine_body,
        grid=grid,
        in_specs=[mat, mat, vec, vec, vec, vec, mat,
                  pl.BlockSpec((C,), lambda i: (0,))],
        out_specs=mat,
        out_shape=jax.ShapeDtypeStruct((N_PAD, C), jnp.float32),
    )(acc0, acc1, den0, den1, asrc, adst, xs, bias)


def kernel(x, edge_index, idx, W, att_src, att_dst, bias):
    sign = jnp.where(idx == 1, jnp.float32(-1.0), jnp.float32(1.0))
    vs = (sign * att_src).reshape(C).astype(jnp.float32)
    vd = (sign * att_dst).reshape(C).astype(jnp.float32)
    x_pad = jnp.concatenate(
        [x, jnp.zeros((N_PAD - N_NODES, D), jnp.float32)], axis=0)
    edges = jnp.concatenate(
        [edge_index, jnp.zeros((2, E_PAD - E), edge_index.dtype)], axis=1)

    xs, asrc, adst = _prep(x_pad, W.T, vs, vd)
    acc, den = _sc_edges(xs, asrc, adst, edges)
    out = _combine(acc[0], acc[1], den[0], den[1], asrc, adst, xs, bias)
    return out[:N_NODES]


# speedup vs baseline: 1.5257x; 1.5257x over previous
import functools

import jax
import jax.numpy as jnp
from jax import lax
from jax.experimental import pallas as pl
from jax.experimental.pallas import tpu as pltpu
from jax.experimental.pallas import tpu_sc as plsc

N_NODES = 10000
N_PAD = 10240
D = 128
C = 128
E = 320000
NC, NS = 2, 16
NW = NC * NS
CHUNK = 64
NCH = 158
E_PAD = (NCH + 1) * NW * CHUNK
ROWS_PER_TILE = N_PAD // NS
WCH = 128
BLK = 1024


def _prep_body(x_ref, wt_ref, vs_ref, vd_ref, xs_ref, asrc_ref, adst_ref):
    xs = jnp.dot(x_ref[...], wt_ref[...], preferred_element_type=jnp.float32)
    xs_ref[...] = xs
    asrc_ref[...] = jnp.sum(xs * vs_ref[...][None, :], axis=1)
    adst_ref[...] = jnp.sum(xs * vd_ref[...][None, :], axis=1)


def _prep(x_pad, wt, vs, vd):
    grid = (N_PAD // BLK,)
    return pl.pallas_call(
        _prep_body,
        grid=grid,
        in_specs=[
            pl.BlockSpec((BLK, D), lambda i: (i, 0)),
            pl.BlockSpec((D, C), lambda i: (0, 0)),
            pl.BlockSpec((C,), lambda i: (0,)),
            pl.BlockSpec((C,), lambda i: (0,)),
        ],
        out_specs=[
            pl.BlockSpec((BLK, C), lambda i: (i, 0)),
            pl.BlockSpec((BLK,), lambda i: (i,)),
            pl.BlockSpec((BLK,), lambda i: (i,)),
        ],
        out_shape=[
            jax.ShapeDtypeStruct((N_PAD, C), jnp.float32),
            jax.ShapeDtypeStruct((N_PAD,), jnp.float32),
            jax.ShapeDtypeStruct((N_PAD,), jnp.float32),
        ],
    )(x_pad, wt, vs, vd)


def _sc_body(xs_hbm, asrc_hbm, adst_hbm, edges_hbm,
             acc_out, den_out,
             asrc_v, adst_v, idx0_v, idx1_v, idx2_v, e0_v, e1_v, e2_v,
             rows0_v, rows1_v, rows2_v, acc_sh, den_sh,
             sem_g0, sem_g1, sem_g2, sem_s0, sem_s1, sem_s2):
    cid = lax.axis_index("c")
    sid = lax.axis_index("s")
    wid = cid * NS + sid
    idx2 = (idx0_v, idx1_v, idx2_v)
    ev = (e0_v, e1_v, e2_v)
    rows = (rows0_v, rows1_v, rows2_v)
    sem_g = (sem_g0, sem_g1, sem_g2)
    sem_s = (sem_s0, sem_s1, sem_s2)

    pltpu.sync_copy(asrc_hbm, asrc_v)
    pltpu.sync_copy(adst_hbm, adst_v)

    zeros16 = jnp.zeros((16,), jnp.float32)

    def _zrow(i, carry):
        for q in range(C // 16):
            rows0_v[i, pl.ds(q * 16, 16)] = zeros16
        return carry

    lax.fori_loop(0, CHUNK, _zrow, 0)
    for q in range(CHUNK // 16):
        e0_v[pl.ds(q * 16, 16)] = zeros16
    for t in range(ROWS_PER_TILE // CHUNK):
        r0 = sid * ROWS_PER_TILE + t * CHUNK
        pltpu.sync_copy(rows0_v, acc_sh.at[pl.ds(r0, CHUNK)])
        pltpu.sync_copy(e0_v, den_sh.at[pl.ds(r0, CHUNK)])
    plsc.subcore_barrier()

    def _issue_gather(b, g):
        base = (wid + NW * g) * CHUNK
        pltpu.sync_copy(edges_hbm.at[:, pl.ds(base, CHUNK)], idx2[b])
        pltpu.async_copy(xs_hbm.at[idx2[b].at[0]], rows[b], sem_g[b])

    def _wait_gather(b):
        pltpu.make_async_copy(
            xs_hbm.at[idx2[b].at[0]], rows[b], sem_g[b]).wait()

    def _wait_scatter(b):
        pltpu.make_async_copy(
            rows[b], acc_sh.at[idx2[b].at[1]], sem_s[b]).wait()
        pltpu.make_async_copy(
            ev[b], den_sh.at[idx2[b].at[1]], sem_s[b]).wait()

    def _process(b):
        for g in range(CHUNK // 16):
            sl = pl.ds(g * 16, 16)
            si = idx2[b][0, sl]
            di = idx2[b][1, sl]
            s = plsc.load_gather(asrc_v, [si]) + plsc.load_gather(adst_v, [di])
            s = jnp.where(s >= 0.0, s, 0.2 * s)
            ev[b][sl] = jnp.where(si != di, jnp.exp(s), 0.0)

        def _scale(it, carry):
            for u in range(4):
                i = it * 4 + u
                spl = plsc.load_gather(
                    ev[b], [jnp.full((16,), 0, jnp.int32) + i])
                for q in range(C // 16):
                    sl = pl.ds(q * 16, 16)
                    rows[b][i, sl] = rows[b][i, sl] * spl
            return carry

        lax.fori_loop(0, CHUNK // 4, _scale, 0)

    def _issue_scatter(b):
        pltpu.async_copy(rows[b], acc_sh.at[idx2[b].at[1]], sem_s[b], add=True)
        pltpu.async_copy(ev[b], den_sh.at[idx2[b].at[1]], sem_s[b], add=True)

    def _iter(b, g, wait_s):
        nb = (b + 1) % 3
        if wait_s:
            _wait_scatter(nb)
        _issue_gather(nb, g + 1)
        _wait_gather(b)
        _process(b)
        _issue_scatter(b)

    _issue_gather(0, 0)
    _iter(0, 0, wait_s=False)
    _iter(1, 1, wait_s=False)

    def _trip(jj, carry):
        _iter(2, 3 * jj + 2, wait_s=True)
        _iter(0, 3 * jj + 3, wait_s=True)
        _iter(1, 3 * jj + 4, wait_s=True)
        return carry

    lax.fori_loop(0, (NCH - 5) // 3 + 1, _trip, 0)
    _wait_gather(2)
    _wait_scatter(0)
    _wait_scatter(1)
    plsc.subcore_barrier()

    for t in range(ROWS_PER_TILE // CHUNK):
        r0 = sid * ROWS_PER_TILE + t * CHUNK
        rv = rows[t % 3]
        pltpu.sync_copy(acc_sh.at[pl.ds(r0, CHUNK)], rv)
        pltpu.sync_copy(rv, acc_out.at[cid, pl.ds(r0, CHUNK)])
    for t in range(ROWS_PER_TILE // C):
        r0 = sid * ROWS_PER_TILE + t * C
        pltpu.sync_copy(den_sh.at[pl.ds(r0, C)], rows0_v.at[0])
        pltpu.sync_copy(rows0_v.at[0], den_out.at[cid, pl.ds(r0, C)])


_sc_edges = functools.partial(
    pl.kernel,
    out_type=[
        jax.ShapeDtypeStruct((NC, N_PAD, C), jnp.float32),
        jax.ShapeDtypeStruct((NC, N_PAD), jnp.float32),
    ],
    mesh=plsc.VectorSubcoreMesh(core_axis_name="c", subcore_axis_name="s"),
    compiler_params=pltpu.CompilerParams(
        needs_layout_passes=False, use_tc_tiling_on_sc=False),
    scratch_types=[
        pltpu.VMEM((N_PAD,), jnp.float32),
        pltpu.VMEM((N_PAD,), jnp.float32),
        pltpu.VMEM((2, CHUNK), jnp.int32),
        pltpu.VMEM((2, CHUNK), jnp.int32),
        pltpu.VMEM((2, CHUNK), jnp.int32),
        pltpu.VMEM((CHUNK,), jnp.float32),
        pltpu.VMEM((CHUNK,), jnp.float32),
        pltpu.VMEM((CHUNK,), jnp.float32),
        pltpu.VMEM((CHUNK, C), jnp.float32),
        pltpu.VMEM((CHUNK, C), jnp.float32),
        pltpu.VMEM((CHUNK, C), jnp.float32),
        pltpu.VMEM_SHARED((N_PAD, C), jnp.float32),
        pltpu.VMEM_SHARED((N_PAD,), jnp.float32),
        pltpu.SemaphoreType.DMA,
        pltpu.SemaphoreType.DMA,
        pltpu.SemaphoreType.DMA,
        pltpu.SemaphoreType.DMA,
        pltpu.SemaphoreType.DMA,
        pltpu.SemaphoreType.DMA,
    ],
)(_sc_body)


def _combine_body(acc0_ref, acc1_ref, den0_ref, den1_ref, asrc_ref, adst_ref,
                  xs_ref, bias_ref, out_ref):
    s = asrc_ref[...] + adst_ref[...]
    s = jnp.where(s >= 0.0, s, 0.2 * s)
    es = jnp.exp(s)
    den = den0_ref[...] + den1_ref[...] + es
    num = acc0_ref[...] + acc1_ref[...] + es[:, None] * xs_ref[...]
    out_ref[...] = num / (den[:, None] + 1e-16) + bias_ref[...][None, :]


def _combine(acc0, acc1, den0, den1, asrc, adst, xs, bias):
    grid = (N_PAD // BLK,)
    mat = pl.BlockSpec((BLK, C), lambda i: (i, 0))
    vec = pl.BlockSpec((BLK,), lambda i: (i,))
    return pl.pallas_call(
        _combine_body,
        grid=grid,
        in_specs=[mat, mat, vec, vec, vec, vec, mat,
                  pl.BlockSpec((C,), lambda i: (0,))],
        out_specs=mat,
        out_shape=jax.ShapeDtypeStruct((N_PAD, C), jnp.float32),
    )(acc0, acc1, den0, den1, asrc, adst, xs, bias)


def kernel(x, edge_index, idx, W, att_src, att_dst, bias):
    sign = jnp.where(idx == 1, jnp.float32(-1.0), jnp.float32(1.0))
    vs = (sign * att_src).reshape(C).astype(jnp.float32)
    vd = (sign * att_dst).reshape(C).astype(jnp.float32)
    x_pad = jnp.concatenate(
        [x, jnp.zeros((N_PAD - N_NODES, D), jnp.float32)], axis=0)
    edges = jnp.concatenate(
        [edge_index, jnp.zeros((2, E_PAD - E), edge_index.dtype)], axis=1)

    xs, asrc, adst = _prep(x_pad, W.T, vs, vd)
    acc, den = _sc_edges(xs, asrc, adst, edges)
    out = _combine(acc[0], acc[1], den[0], den[1], asrc, adst, xs, bias)
    return out[:N_NODES]
